# baseline (speedup 1.0000x reference)
import functools

import jax
import jax.numpy as jnp
from jax import lax
from jax.experimental import pallas as pl
from jax.experimental.pallas import tpu as pltpu
from jax.experimental.pallas import tpu_sc as plsc

N = 10000
D = 128
E = 320000
H = 10000
P = 32
SCALE = 1.0 / (P ** 0.5)

NC = 2
NS = 16
NW = NC * NS
G = 128
GPW = 80
E_PAD = NW * GPW * G
TG = NW * GPW
RT = 640
H_PAD = NS * RT
NB1 = 8
NB2 = 5

NB = 1000
HB = 1280
NGB = H_PAD // HB

_SC_PARAMS = pltpu.CompilerParams(
    use_tc_tiling_on_sc=False, needs_layout_passes=False,
    disable_bounds_checks=True)


def _proj_body(x_ref, wk_ref, wv_ref, k_ref, kv_ref, m_ref):
    i = pl.program_id(0)
    x = x_ref[...]
    k = lax.dot_general(x, wk_ref[...], (((1,), (1,)), ((), ())),
                        preferred_element_type=jnp.float32)
    v = lax.dot_general(x, wv_ref[...], (((1,), (1,)), ((), ())),
                        preferred_element_type=jnp.float32)
    k_ref[...] = k
    kv_ref[...] = jnp.concatenate([k, v], axis=1)
    mb = jnp.full((1, 128), jnp.max(jnp.sum(k * k, axis=1)))

    @pl.when(i == 0)
    def _():
        m_ref[...] = mb

    @pl.when(i > 0)
    def _():
        m_ref[...] = jnp.maximum(m_ref[...], mb)


_proj = pl.pallas_call(
    _proj_body,
    grid=(N // NB,),
    in_specs=[
        pl.BlockSpec((NB, D), lambda i: (i, 0)),
        pl.BlockSpec((P, D), lambda i: (0, 0)),
        pl.BlockSpec((P, D), lambda i: (0, 0)),
    ],
    out_specs=[
        pl.BlockSpec((NB, P), lambda i: (i, 0)),
        pl.BlockSpec((NB, 2 * P), lambda i: (i, 0)),
        pl.BlockSpec((1, 128), lambda i: (0, 0)),
    ],
    out_shape=[
        jax.ShapeDtypeStruct((N, P), jnp.float32),
        jax.ShapeDtypeStruct((N, 2 * P), jnp.float32),
        jax.ShapeDtypeStruct((1, 128), jnp.float32),
    ],
)


def _sc_pass1_body(k_hbm, nidx_hbm, hidx_hbm, z2d_hbm, z1d_hbm, ones_hbm,
                   ks_out, cnt_out,
                   ks_sp, cnt_sp, nidx_all, hidx_all, ones_v, st1d_v,
                   kr0, kr1, kr2, kr3, kr4, kr5, kr6, kr7,
                   gs0, gs1, gs2, gs3, gs4, gs5, gs6, gs7,
                   ss0, ss1, ss2, ss3, ss4, ss5, ss6, ss7):
    kr = [kr0, kr1, kr2, kr3, kr4, kr5, kr6, kr7]
    gsem = [gs0, gs1, gs2, gs3, gs4, gs5, gs6, gs7]
    ssem = [ss0, ss1, ss2, ss3, ss4, ss5, ss6, ss7]
    c = lax.axis_index("c")
    t = lax.axis_index("s")
    w = t * NC + c
    pltpu.sync_copy(nidx_hbm.at[pl.ds(w * GPW, GPW)], nidx_all)
    pltpu.sync_copy(hidx_hbm.at[pl.ds(w * GPW, GPW)], hidx_all)
    pltpu.sync_copy(ones_hbm, ones_v)
    for b in range(NB1):
        pltpu.async_copy(k_hbm.at[nidx_all.at[b]], kr[b], gsem[b])
    pltpu.sync_copy(z1d_hbm, st1d_v)
    for ch in range(RT // G):
        pltpu.sync_copy(z2d_hbm, ks_sp.at[pl.ds(t * RT + ch * G, G)])
        pltpu.sync_copy(st1d_v, cnt_sp.at[pl.ds(t * RT + ch * G, G)])
    plsc.subcore_barrier()

    nsup = GPW // NB1

    def body(i, carry):
        for b in range(NB1):
            g = i * NB1 + b
            pltpu.make_async_copy(k_hbm.at[nidx_all.at[g]], kr[b],
                                  gsem[b]).wait()
            pltpu.async_copy(kr[b], ks_sp.at[hidx_all.at[g]], ssem[b],
                             add=True)
            pltpu.async_copy(ones_v, cnt_sp.at[hidx_all.at[g]], ssem[b],
                             add=True)
        for b in range(NB1):
            g = i * NB1 + b
            pltpu.make_async_copy(kr[b], ks_sp.at[hidx_all.at[g]],
                                  ssem[b]).wait()
            pltpu.make_async_copy(ones_v, cnt_sp.at[hidx_all.at[g]],
                                  ssem[b]).wait()

            @pl.when(i < nsup - 1)
            def _(b=b, g=g):
                pltpu.async_copy(k_hbm.at[nidx_all.at[g + NB1]], kr[b],
                                 gsem[b])
        return carry

    lax.fori_loop(0, nsup, body, 0)
    plsc.subcore_barrier()
    for ch in range(RT // G):
        sl = pl.ds(t * RT + ch * G, G)
        pltpu.sync_copy(ks_sp.at[sl], kr[0])
        pltpu.sync_copy(kr[0], ks_out.at[c, sl])
        pltpu.sync_copy(cnt_sp.at[sl], st1d_v)
        pltpu.sync_copy(st1d_v, cnt_out.at[c, sl])


_sc_pass1 = functools.partial(
    pl.kernel,
    out_type=(
        jax.ShapeDtypeStruct((NC, H_PAD, P), jnp.float32),
        jax.ShapeDtypeStruct((NC, H_PAD), jnp.float32),
    ),
    mesh=plsc.VectorSubcoreMesh(core_axis_name="c", subcore_axis_name="s"),
    compiler_params=_SC_PARAMS,
    scratch_types=(
        [pltpu.VMEM_SHARED((H_PAD, P), jnp.float32),
         pltpu.VMEM_SHARED((H_PAD,), jnp.float32),
         pltpu.VMEM((GPW, G), jnp.int32),
         pltpu.VMEM((GPW, G), jnp.int32),
         pltpu.VMEM((G,), jnp.float32),
         pltpu.VMEM((G,), jnp.float32)]
        + [pltpu.VMEM((G, P), jnp.float32)] * NB1
        + [pltpu.SemaphoreType.DMA] * (2 * NB1)
    ),
)(_sc_pass1_body)


def _cent_body(ks_ref, cnt_ref, knm_ref, c_ref, g_ref):
    i = pl.program_id(0)
    ks = ks_ref[0] + ks_ref[1]
    cnt = cnt_ref[0] + cnt_ref[1]
    c = ks / jnp.maximum(cnt, 1.0)[:, None]
    c_ref[...] = c
    mb = jnp.full((1, 128), jnp.max(jnp.sum(c * c, axis=1)))

    @pl.when(i == 0)
    def _():
        g_ref[...] = mb

    @pl.when(jnp.logical_and(i > 0, i < NGB - 1))
    def _():
        g_ref[...] = jnp.maximum(g_ref[...], mb)

    @pl.when(i == NGB - 1)
    def _():
        maxc2 = jnp.maximum(g_ref[...], mb)
        maxk2 = jnp.max(knm_ref[...])
        g_ref[...] = SCALE * (jnp.sqrt(maxc2) + jnp.sqrt(maxk2))


_cent = pl.pallas_call(
    _cent_body,
    grid=(NGB,),
    in_specs=[
        pl.BlockSpec((NC, HB, P), lambda i: (0, i, 0)),
        pl.BlockSpec((NC, HB), lambda i: (0, i)),
        pl.BlockSpec((1, 128), lambda i: (0, 0)),
    ],
    out_specs=[
        pl.BlockSpec((HB, P), lambda i: (i, 0)),
        pl.BlockSpec((1, 128), lambda i: (0, 0)),
    ],
    out_shape=[
        jax.ShapeDtypeStruct((H_PAD, P), jnp.float32),
        jax.ShapeDtypeStruct((1, 128), jnp.float32),
    ],
)


def _sqrt16(x):
    i = plsc.bitcast(x, jnp.int32)
    i = jnp.int32(0x5F3759DF) - (i >> 1)
    y = plsc.bitcast(i, jnp.float32)
    for _ in range(3):
        y = y * (1.5 - 0.5 * x * y * y)
    return jnp.where(x > 0, x * y, 0.0)


def _sc_pass2_body(kv_hbm, c_hbm, g_hbm, nidx_hbm, hidx_hbm, z2d_hbm, z1d_hbm,
                   s_out, den_out,
                   s_sp, den_sp, nidx_all, hidx_all, g_v, st1d_v,
                   kv0, kv1, kv2, kv3, kv4, cr0, cr1, cr2, cr3, cr4,
                   wv0, wv1, wv2, wv3, wv4, ex0, ex1, ex2, ex3, ex4,
                   gk0, gk1, gk2, gk3, gk4, gc0, gc1, gc2, gc3, gc4,
                   ss0, ss1, ss2, ss3, ss4):
    kvr = [kv0, kv1, kv2, kv3, kv4]
    crr = [cr0, cr1, cr2, cr3, cr4]
    wvr = [wv0, wv1, wv2, wv3, wv4]
    exr = [ex0, ex1, ex2, ex3, ex4]
    gksem = [gk0, gk1, gk2, gk3, gk4]
    gcsem = [gc0, gc1, gc2, gc3, gc4]
    ssem = [ss0, ss1, ss2, ss3, ss4]
    c = lax.axis_index("c")
    t = lax.axis_index("s")
    w = t * NC + c
    pltpu.sync_copy(nidx_hbm.at[pl.ds(w * GPW, GPW)], nidx_all)
    pltpu.sync_copy(hidx_hbm.at[pl.ds(w * GPW, GPW)], hidx_all)
    pltpu.sync_copy(g_hbm, g_v)
    for b in range(NB2):
        pltpu.async_copy(kv_hbm.at[nidx_all.at[b]], kvr[b], gksem[b])
        pltpu.async_copy(c_hbm.at[hidx_all.at[b]], crr[b], gcsem[b])
    pltpu.sync_copy(z1d_hbm, st1d_v)
    for ch in range(RT // G):
        pltpu.sync_copy(z2d_hbm, s_sp.at[pl.ds(t * RT + ch * G, G)])
        pltpu.sync_copy(st1d_v, den_sp.at[pl.ds(t * RT + ch * G, G)])
    plsc.subcore_barrier()

    gvec = g_v[...]
    rows0 = jnp.arange(16, dtype=jnp.int32)
    nsup = GPW // NB2

    def body(i, carry):
        for b in range(NB2):
            g = i * NB2 + b
            pltpu.make_async_copy(kv_hbm.at[nidx_all.at[g]], kvr[b],
                                  gksem[b]).wait()
            pltpu.make_async_copy(c_hbm.at[hidx_all.at[g]], crr[b],
                                  gcsem[b]).wait()

            @pl.when(i > 0)
            def _(b=b, g=g):
                pltpu.make_async_copy(wvr[b], s_sp.at[hidx_all.at[g - NB2]],
                                      ssem[b]).wait()
                pltpu.make_async_copy(exr[b], den_sp.at[hidx_all.at[g - NB2]],
                                      ssem[b]).wait()

            def s8_body(s8, cc2, _kv=kvr[b], _cr=crr[b], _wv=wvr[b],
                        _ex=exr[b]):
                rows = rows0 + s8 * 16
                accs = [jnp.zeros((16,), jnp.float32) for _ in range(4)]
                for j in range(P):
                    colj = (rows0 + j) & (P - 1)
                    kvx = plsc.load_gather(_kv, [rows, colj])
                    cvx = plsc.load_gather(_cr, [rows, colj])
                    dlt = kvx - cvx
                    accs[j % 4] = accs[j % 4] + dlt * dlt
                acc = (accs[0] + accs[1]) + (accs[2] + accs[3])
                ex = jnp.exp(_sqrt16(acc) * SCALE - gvec)
                _ex[pl.ds(s8 * 16, 16)] = ex
                for j in range(P):
                    colj = (rows0 + j) & (P - 1)
                    vv = plsc.load_gather(_kv, [rows, colj + P])
                    plsc.store_scatter(_wv, [rows, colj], vv * ex)
                return cc2

            lax.fori_loop(0, G // 16, s8_body, 0)

            @pl.when(i < nsup - 1)
            def _(b=b, g=g):
                pltpu.async_copy(kv_hbm.at[nidx_all.at[g + NB2]], kvr[b],
                                 gksem[b])
                pltpu.async_copy(c_hbm.at[hidx_all.at[g + NB2]], crr[b],
                                 gcsem[b])

            pltpu.async_copy(wvr[b], s_sp.at[hidx_all.at[g]], ssem[b],
                             add=True)
            pltpu.async_copy(exr[b], den_sp.at[hidx_all.at[g]], ssem[b],
                             add=True)
        return carry

    lax.fori_loop(0, nsup, body, 0)
    for b in range(NB2):
        g = GPW - NB2 + b
        pltpu.make_async_copy(wvr[b], s_sp.at[hidx_all.at[g]],
                              ssem[b]).wait()
        pltpu.make_async_copy(exr[b], den_sp.at[hidx_all.at[g]],
                              ssem[b]).wait()
    plsc.subcore_barrier()
    for ch in range(RT // G):
        sl = pl.ds(t * RT + ch * G, G)
        pltpu.sync_copy(s_sp.at[sl], wv0)
        pltpu.sync_copy(wv0, s_out.at[c, sl])
        pltpu.sync_copy(den_sp.at[sl], st1d_v)
        pltpu.sync_copy(st1d_v, den_out.at[c, sl])


_sc_pass2 = functools.partial(
    pl.kernel,
    out_type=(
        jax.ShapeDtypeStruct((NC, H_PAD, P), jnp.float32),
        jax.ShapeDtypeStruct((NC, H_PAD), jnp.float32),
    ),
    mesh=plsc.VectorSubcoreMesh(core_axis_name="c", subcore_axis_name="s"),
    compiler_params=_SC_PARAMS,
    scratch_types=(
        [pltpu.VMEM_SHARED((H_PAD, P), jnp.float32),
         pltpu.VMEM_SHARED((H_PAD,), jnp.float32),
         pltpu.VMEM((GPW, G), jnp.int32),
         pltpu.VMEM((GPW, G), jnp.int32),
         pltpu.VMEM((16,), jnp.float32),
         pltpu.VMEM((G,), jnp.float32)]
        + [pltpu.VMEM((G, 2 * P), jnp.float32)] * NB2
        + [pltpu.VMEM((G, P), jnp.float32)] * (2 * NB2)
        + [pltpu.VMEM((G,), jnp.float32)] * NB2
        + [pltpu.SemaphoreType.DMA] * (3 * NB2)
    ),
)(_sc_pass2_body)


def _final_body(s_ref, den_ref, wv_ref, o_ref):
    s = s_ref[0] + s_ref[1]
    den = den_ref[0] + den_ref[1]
    hf = s / jnp.where(den > 0, den, 1.0)[:, None]
    o_ref[...] = jnp.dot(hf, wv_ref[...], preferred_element_type=jnp.float32)


_final = pl.pallas_call(
    _final_body,
    grid=(NGB,),
    in_specs=[
        pl.BlockSpec((NC, HB, P), lambda i: (0, i, 0)),
        pl.BlockSpec((NC, HB), lambda i: (0, i)),
        pl.BlockSpec((P, D), lambda i: (0, 0)),
    ],
    out_specs=pl.BlockSpec((HB, D), lambda i: (i, 0)),
    out_shape=jax.ShapeDtypeStruct((H_PAD, D), jnp.float32),
)


def kernel(node_feats, hyperedge_index, num_hyperedges, W_key, W_value):
    del num_hyperedges
    nidx = hyperedge_index[0].astype(jnp.int32)
    hidx = hyperedge_index[1].astype(jnp.int32)
    pad = E_PAD - E
    spread = jnp.arange(pad, dtype=jnp.int32)
    nidx_p = jnp.concatenate([nidx, spread % N])
    hidx_p = jnp.concatenate([hidx, H + spread % (H_PAD - H)])
    nidx2d = nidx_p.reshape(TG, G)
    hidx2d = hidx_p.reshape(TG, G)
    z2d = jnp.zeros((G, P), jnp.float32)
    z1d = jnp.zeros((G,), jnp.float32)
    ones = jnp.ones((G,), jnp.float32)

    k_arr, kv_arr, knm = _proj(node_feats, W_key, W_value)
    ks2, cnt2 = _sc_pass1(k_arr, nidx2d, hidx2d, z2d, z1d, ones)
    c_arr, gfull = _cent(ks2, cnt2, knm)
    g16 = gfull[0, :16]
    s2, den2 = _sc_pass2(kv_arr, c_arr, g16, nidx2d, hidx2d, z2d, z1d)
    out_pad = _final(s2, den2, W_value)
    return out_pad[:H]


# speedup vs baseline: 1.0040x; 1.0040x over previous
import functools

import jax
import jax.numpy as jnp
from jax import lax
from jax.experimental import pallas as pl
from jax.experimental.pallas import tpu as pltpu
from jax.experimental.pallas import tpu_sc as plsc

N = 10000
D = 128
E = 320000
H = 10000
P = 32
SCALE = 1.0 / (P ** 0.5)

NC = 2
NS = 16
NW = NC * NS
G = 128
GPW = 80
E_PAD = NW * GPW * G
TG = NW * GPW
RT = 640
H_PAD = NS * RT
NB1 = 8
NB2 = 4

NB = 1000
HB = 1280
NGB = H_PAD // HB

_SC_PARAMS = pltpu.CompilerParams(
    use_tc_tiling_on_sc=False, needs_layout_passes=False,
    disable_bounds_checks=True)


def _proj_body(x_ref, wk_ref, wv_ref, k_ref, kv_ref, m_ref):
    i = pl.program_id(0)
    x = x_ref[...]
    k = lax.dot_general(x, wk_ref[...], (((1,), (1,)), ((), ())),
                        preferred_element_type=jnp.float32)
    v = lax.dot_general(x, wv_ref[...], (((1,), (1,)), ((), ())),
                        preferred_element_type=jnp.float32)
    k_ref[...] = k
    kv_ref[...] = jnp.concatenate([k, v], axis=1)
    mb = jnp.full((1, 128), jnp.max(jnp.sum(k * k, axis=1)))

    @pl.when(i == 0)
    def _():
        m_ref[...] = mb

    @pl.when(i > 0)
    def _():
        m_ref[...] = jnp.maximum(m_ref[...], mb)


_proj = pl.pallas_call(
    _proj_body,
    grid=(N // NB,),
    in_specs=[
        pl.BlockSpec((NB, D), lambda i: (i, 0)),
        pl.BlockSpec((P, D), lambda i: (0, 0)),
        pl.BlockSpec((P, D), lambda i: (0, 0)),
    ],
    out_specs=[
        pl.BlockSpec((NB, P), lambda i: (i, 0)),
        pl.BlockSpec((NB, 2 * P), lambda i: (i, 0)),
        pl.BlockSpec((1, 128), lambda i: (0, 0)),
    ],
    out_shape=[
        jax.ShapeDtypeStruct((N, P), jnp.float32),
        jax.ShapeDtypeStruct((N, 2 * P), jnp.float32),
        jax.ShapeDtypeStruct((1, 128), jnp.float32),
    ],
)


def _sc_pass1_body(k_hbm, nidx_hbm, hidx_hbm, z2d_hbm, z1d_hbm, ones_hbm,
                   ks_out, cnt_out,
                   ks_sp, cnt_sp, nidx_all, hidx_all, ones_v, st1d_v,
                   kr0, kr1, kr2, kr3, kr4, kr5, kr6, kr7,
                   gs0, gs1, gs2, gs3, gs4, gs5, gs6, gs7,
                   ss0, ss1, ss2, ss3, ss4, ss5, ss6, ss7):
    kr = [kr0, kr1, kr2, kr3, kr4, kr5, kr6, kr7]
    gsem = [gs0, gs1, gs2, gs3, gs4, gs5, gs6, gs7]
    ssem = [ss0, ss1, ss2, ss3, ss4, ss5, ss6, ss7]
    c = lax.axis_index("c")
    t = lax.axis_index("s")
    w = t * NC + c
    pltpu.sync_copy(nidx_hbm.at[pl.ds(w * GPW, GPW)], nidx_all)
    pltpu.sync_copy(hidx_hbm.at[pl.ds(w * GPW, GPW)], hidx_all)
    pltpu.sync_copy(ones_hbm, ones_v)
    for b in range(NB1):
        pltpu.async_copy(k_hbm.at[nidx_all.at[b]], kr[b], gsem[b])
    pltpu.sync_copy(z1d_hbm, st1d_v)
    for ch in range(RT // G):
        pltpu.sync_copy(z2d_hbm, ks_sp.at[pl.ds(t * RT + ch * G, G)])
        pltpu.sync_copy(st1d_v, cnt_sp.at[pl.ds(t * RT + ch * G, G)])
    plsc.subcore_barrier()

    nsup = GPW // NB1

    def body(i, carry):
        for b in range(NB1):
            g = i * NB1 + b
            pltpu.make_async_copy(k_hbm.at[nidx_all.at[g]], kr[b],
                                  gsem[b]).wait()
            pltpu.async_copy(kr[b], ks_sp.at[hidx_all.at[g]], ssem[b],
                             add=True)
            pltpu.async_copy(ones_v, cnt_sp.at[hidx_all.at[g]], ssem[b],
                             add=True)
        for b in range(NB1):
            g = i * NB1 + b
            pltpu.make_async_copy(kr[b], ks_sp.at[hidx_all.at[g]],
                                  ssem[b]).wait()
            pltpu.make_async_copy(ones_v, cnt_sp.at[hidx_all.at[g]],
                                  ssem[b]).wait()

            @pl.when(i < nsup - 1)
            def _(b=b, g=g):
                pltpu.async_copy(k_hbm.at[nidx_all.at[g + NB1]], kr[b],
                                 gsem[b])
        return carry

    lax.fori_loop(0, nsup, body, 0)
    plsc.subcore_barrier()
    for ch in range(RT // G):
        sl = pl.ds(t * RT + ch * G, G)
        pltpu.sync_copy(ks_sp.at[sl], kr[0])
        pltpu.sync_copy(kr[0], ks_out.at[c, sl])
        pltpu.sync_copy(cnt_sp.at[sl], st1d_v)
        pltpu.sync_copy(st1d_v, cnt_out.at[c, sl])


_sc_pass1 = functools.partial(
    pl.kernel,
    out_type=(
        jax.ShapeDtypeStruct((NC, H_PAD, P), jnp.float32),
        jax.ShapeDtypeStruct((NC, H_PAD), jnp.float32),
    ),
    mesh=plsc.VectorSubcoreMesh(core_axis_name="c", subcore_axis_name="s"),
    compiler_params=_SC_PARAMS,
    scratch_types=(
        [pltpu.VMEM_SHARED((H_PAD, P), jnp.float32),
         pltpu.VMEM_SHARED((H_PAD,), jnp.float32),
         pltpu.VMEM((GPW, G), jnp.int32),
         pltpu.VMEM((GPW, G), jnp.int32),
         pltpu.VMEM((G,), jnp.float32),
         pltpu.VMEM((G,), jnp.float32)]
        + [pltpu.VMEM((G, P), jnp.float32)] * NB1
        + [pltpu.SemaphoreType.DMA] * (2 * NB1)
    ),
)(_sc_pass1_body)


def _cent_body(ks_ref, cnt_ref, knm_ref, c_ref, g_ref):
    i = pl.program_id(0)
    ks = ks_ref[0] + ks_ref[1]
    cnt = cnt_ref[0] + cnt_ref[1]
    c = ks / jnp.maximum(cnt, 1.0)[:, None]
    c_ref[...] = c
    mb = jnp.full((1, 128), jnp.max(jnp.sum(c * c, axis=1)))

    @pl.when(i == 0)
    def _():
        g_ref[...] = mb

    @pl.when(jnp.logical_and(i > 0, i < NGB - 1))
    def _():
        g_ref[...] = jnp.maximum(g_ref[...], mb)

    @pl.when(i == NGB - 1)
    def _():
        maxc2 = jnp.maximum(g_ref[...], mb)
        maxk2 = jnp.max(knm_ref[...])
        g_ref[...] = SCALE * (jnp.sqrt(maxc2) + jnp.sqrt(maxk2))


_cent = pl.pallas_call(
    _cent_body,
    grid=(NGB,),
    in_specs=[
        pl.BlockSpec((NC, HB, P), lambda i: (0, i, 0)),
        pl.BlockSpec((NC, HB), lambda i: (0, i)),
        pl.BlockSpec((1, 128), lambda i: (0, 0)),
    ],
    out_specs=[
        pl.BlockSpec((HB, P), lambda i: (i, 0)),
        pl.BlockSpec((1, 128), lambda i: (0, 0)),
    ],
    out_shape=[
        jax.ShapeDtypeStruct((H_PAD, P), jnp.float32),
        jax.ShapeDtypeStruct((1, 128), jnp.float32),
    ],
)


def _sqrt16(x):
    i = plsc.bitcast(x, jnp.int32)
    i = jnp.int32(0x5F3759DF) - (i >> 1)
    y = plsc.bitcast(i, jnp.float32)
    for _ in range(3):
        y = y * (1.5 - 0.5 * x * y * y)
    return jnp.where(x > 0, x * y, 0.0)


def _sc_pass2_body(kv_hbm, c_hbm, g_hbm, nidx_hbm, hidx_hbm, z2d_hbm, z1d_hbm,
                   s_out, den_out,
                   s_sp, den_sp, nidx_all, hidx_all, g_v, st1d_v,
                   kv0, kv1, kv2, kv3, cr0, cr1, cr2, cr3,
                   wv0, wv1, wv2, wv3, ex0, ex1, ex2, ex3,
                   gk0, gk1, gk2, gk3, gc0, gc1, gc2, gc3,
                   ss0, ss1, ss2, ss3):
    kvr = [kv0, kv1, kv2, kv3]
    crr = [cr0, cr1, cr2, cr3]
    wvr = [wv0, wv1, wv2, wv3]
    exr = [ex0, ex1, ex2, ex3]
    gksem = [gk0, gk1, gk2, gk3]
    gcsem = [gc0, gc1, gc2, gc3]
    ssem = [ss0, ss1, ss2, ss3]
    c = lax.axis_index("c")
    t = lax.axis_index("s")
    w = t * NC + c
    pltpu.sync_copy(nidx_hbm.at[pl.ds(w * GPW, GPW)], nidx_all)
    pltpu.sync_copy(hidx_hbm.at[pl.ds(w * GPW, GPW)], hidx_all)
    pltpu.sync_copy(g_hbm, g_v)
    for b in range(NB2):
        pltpu.async_copy(kv_hbm.at[nidx_all.at[b]], kvr[b], gksem[b])
        pltpu.async_copy(c_hbm.at[hidx_all.at[b]], crr[b], gcsem[b])
    pltpu.sync_copy(z1d_hbm, st1d_v)
    for ch in range(RT // G):
        pltpu.sync_copy(z2d_hbm, s_sp.at[pl.ds(t * RT + ch * G, G)])
        pltpu.sync_copy(st1d_v, den_sp.at[pl.ds(t * RT + ch * G, G)])
    plsc.subcore_barrier()

    gvec = g_v[...]
    rows0 = jnp.arange(16, dtype=jnp.int32)
    nsup = GPW // NB2

    def body(i, carry):
        for b in range(NB2):
            g = i * NB2 + b
            pltpu.make_async_copy(kv_hbm.at[nidx_all.at[g]], kvr[b],
                                  gksem[b]).wait()
            pltpu.make_async_copy(c_hbm.at[hidx_all.at[g]], crr[b],
                                  gcsem[b]).wait()

            @pl.when(i > 0)
            def _(b=b, g=g):
                pltpu.make_async_copy(wvr[b], s_sp.at[hidx_all.at[g - NB2]],
                                      ssem[b]).wait()
                pltpu.make_async_copy(exr[b], den_sp.at[hidx_all.at[g - NB2]],
                                      ssem[b]).wait()

            def s8_body(s8, cc2, _kv=kvr[b], _cr=crr[b], _wv=wvr[b],
                        _ex=exr[b]):
                rows = rows0 + s8 * 16
                accs = [jnp.zeros((16,), jnp.float32) for _ in range(4)]
                for j in range(P):
                    colj = (rows0 + j) & (P - 1)
                    kvx = plsc.load_gather(_kv, [rows, colj])
                    cvx = plsc.load_gather(_cr, [rows, colj])
                    dlt = kvx - cvx
                    accs[j % 4] = accs[j % 4] + dlt * dlt
                acc = (accs[0] + accs[1]) + (accs[2] + accs[3])
                ex = jnp.exp(_sqrt16(acc) * SCALE - gvec)
                _ex[pl.ds(s8 * 16, 16)] = ex
                for j in range(P):
                    colj = (rows0 + j) & (P - 1)
                    vv = plsc.load_gather(_kv, [rows, colj + P])
                    plsc.store_scatter(_wv, [rows, colj], vv * ex)
                return cc2

            lax.fori_loop(0, G // 16, s8_body, 0)

            @pl.when(i < nsup - 1)
            def _(b=b, g=g):
                pltpu.async_copy(kv_hbm.at[nidx_all.at[g + NB2]], kvr[b],
                                 gksem[b])
                pltpu.async_copy(c_hbm.at[hidx_all.at[g + NB2]], crr[b],
                                 gcsem[b])

            pltpu.async_copy(wvr[b], s_sp.at[hidx_all.at[g]], ssem[b],
                             add=True)
            pltpu.async_copy(exr[b], den_sp.at[hidx_all.at[g]], ssem[b],
                             add=True)
        return carry

    lax.fori_loop(0, nsup, body, 0)
    for b in range(NB2):
        g = GPW - NB2 + b
        pltpu.make_async_copy(wvr[b], s_sp.at[hidx_all.at[g]],
                              ssem[b]).wait()
        pltpu.make_async_copy(exr[b], den_sp.at[hidx_all.at[g]],
                              ssem[b]).wait()
    plsc.subcore_barrier()
    for ch in range(RT // G):
        sl = pl.ds(t * RT + ch * G, G)
        pltpu.sync_copy(s_sp.at[sl], wv0)
        pltpu.sync_copy(wv0, s_out.at[c, sl])
        pltpu.sync_copy(den_sp.at[sl], st1d_v)
        pltpu.sync_copy(st1d_v, den_out.at[c, sl])


_sc_pass2 = functools.partial(
    pl.kernel,
    out_type=(
        jax.ShapeDtypeStruct((NC, H_PAD, P), jnp.float32),
        jax.ShapeDtypeStruct((NC, H_PAD), jnp.float32),
    ),
    mesh=plsc.VectorSubcoreMesh(core_axis_name="c", subcore_axis_name="s"),
    compiler_params=_SC_PARAMS,
    scratch_types=(
        [pltpu.VMEM_SHARED((H_PAD, P), jnp.float32),
         pltpu.VMEM_SHARED((H_PAD,), jnp.float32),
         pltpu.VMEM((GPW, G), jnp.int32),
         pltpu.VMEM((GPW, G), jnp.int32),
         pltpu.VMEM((16,), jnp.float32),
         pltpu.VMEM((G,), jnp.float32)]
        + [pltpu.VMEM((G, 2 * P), jnp.float32)] * NB2
        + [pltpu.VMEM((G, P), jnp.float32)] * (2 * NB2)
        + [pltpu.VMEM((G,), jnp.float32)] * NB2
        + [pltpu.SemaphoreType.DMA] * (3 * NB2)
    ),
)(_sc_pass2_body)


def _final_body(s_ref, den_ref, wv_ref, o_ref):
    s = s_ref[0] + s_ref[1]
    den = den_ref[0] + den_ref[1]
    hf = s / jnp.where(den > 0, den, 1.0)[:, None]
    o_ref[...] = jnp.dot(hf, wv_ref[...], preferred_element_type=jnp.float32)


_final = pl.pallas_call(
    _final_body,
    grid=(NGB,),
    in_specs=[
        pl.BlockSpec((NC, HB, P), lambda i: (0, i, 0)),
        pl.BlockSpec((NC, HB), lambda i: (0, i)),
        pl.BlockSpec((P, D), lambda i: (0, 0)),
    ],
    out_specs=pl.BlockSpec((HB, D), lambda i: (i, 0)),
    out_shape=jax.ShapeDtypeStruct((H_PAD, D), jnp.float32),
)


def kernel(node_feats, hyperedge_index, num_hyperedges, W_key, W_value):
    del num_hyperedges
    nidx = hyperedge_index[0].astype(jnp.int32)
    hidx = hyperedge_index[1].astype(jnp.int32)
    pad = E_PAD - E
    spread = jnp.arange(pad, dtype=jnp.int32)
    nidx_p = jnp.concatenate([nidx, spread % N])
    hidx_p = jnp.concatenate([hidx, H + spread % (H_PAD - H)])
    nidx2d = nidx_p.reshape(TG, G)
    hidx2d = hidx_p.reshape(TG, G)
    z2d = jnp.zeros((G, P), jnp.float32)
    z1d = jnp.zeros((G,), jnp.float32)
    ones = jnp.ones((G,), jnp.float32)

    k_arr, kv_arr, knm = _proj(node_feats, W_key, W_value)
    ks2, cnt2 = _sc_pass1(k_arr, nidx2d, hidx2d, z2d, z1d, ones)
    c_arr, gfull = _cent(ks2, cnt2, knm)
    g16 = gfull[0, :16]
    s2, den2 = _sc_pass2(kv_arr, c_arr, g16, nidx2d, hidx2d, z2d, z1d)
    out_pad = _final(s2, den2, W_value)
    return out_pad[:H]


# speedup vs baseline: 1.0086x; 1.0045x over previous
import functools

import jax
import jax.numpy as jnp
from jax import lax
from jax.experimental import pallas as pl
from jax.experimental.pallas import tpu as pltpu
from jax.experimental.pallas import tpu_sc as plsc

N = 10000
D = 128
E = 320000
H = 10000
P = 32
SCALE = 1.0 / (P ** 0.5)

NC = 2
NS = 16
NW = NC * NS
G = 128
GPW = 80
E_PAD = NW * GPW * G
TG = NW * GPW
RT = 640
H_PAD = NS * RT
NB1 = 8
NB2 = 4

NB = 1000
HB = 1280
NGB = H_PAD // HB

_SC_PARAMS = pltpu.CompilerParams(
    use_tc_tiling_on_sc=False, needs_layout_passes=False,
    disable_bounds_checks=True)


def _proj_body(x_ref, wk_ref, wv_ref, k_ref, kv_ref, m_ref):
    i = pl.program_id(0)
    x = x_ref[...]
    k = lax.dot_general(x, wk_ref[...], (((1,), (1,)), ((), ())),
                        preferred_element_type=jnp.float32)
    v = lax.dot_general(x, wv_ref[...], (((1,), (1,)), ((), ())),
                        preferred_element_type=jnp.float32)
    k_ref[...] = k
    kv_ref[...] = jnp.concatenate([k, v], axis=1)
    mb = jnp.full((1, 128), jnp.max(jnp.sum(k * k, axis=1)))

    @pl.when(i == 0)
    def _():
        m_ref[...] = mb

    @pl.when(i > 0)
    def _():
        m_ref[...] = jnp.maximum(m_ref[...], mb)


_proj = pl.pallas_call(
    _proj_body,
    grid=(N // NB,),
    in_specs=[
        pl.BlockSpec((NB, D), lambda i: (i, 0)),
        pl.BlockSpec((P, D), lambda i: (0, 0)),
        pl.BlockSpec((P, D), lambda i: (0, 0)),
    ],
    out_specs=[
        pl.BlockSpec((NB, P), lambda i: (i, 0)),
        pl.BlockSpec((NB, 2 * P), lambda i: (i, 0)),
        pl.BlockSpec((1, 128), lambda i: (0, 0)),
    ],
    out_shape=[
        jax.ShapeDtypeStruct((N, P), jnp.float32),
        jax.ShapeDtypeStruct((N, 2 * P), jnp.float32),
        jax.ShapeDtypeStruct((1, 128), jnp.float32),
    ],
)


def _sc_pass1_body(k_hbm, nidx_hbm, hidx_hbm, z2d_hbm, z1d_hbm, ones_hbm,
                   ks_out, cnt_out,
                   ks_sp, cnt_sp, nidx_all, hidx_all, ones_v, st1d_v,
                   kr0, kr1, kr2, kr3, kr4, kr5, kr6, kr7,
                   gs0, gs1, gs2, gs3, gs4, gs5, gs6, gs7,
                   ss0, ss1, ss2, ss3, ss4, ss5, ss6, ss7):
    kr = [kr0, kr1, kr2, kr3, kr4, kr5, kr6, kr7]
    gsem = [gs0, gs1, gs2, gs3, gs4, gs5, gs6, gs7]
    ssem = [ss0, ss1, ss2, ss3, ss4, ss5, ss6, ss7]
    c = lax.axis_index("c")
    t = lax.axis_index("s")
    w = t * NC + c
    pltpu.sync_copy(nidx_hbm.at[pl.ds(w * GPW, GPW)], nidx_all)
    pltpu.sync_copy(hidx_hbm.at[pl.ds(w * GPW, GPW)], hidx_all)
    pltpu.sync_copy(ones_hbm, ones_v)
    for b in range(NB1):
        pltpu.async_copy(k_hbm.at[nidx_all.at[b]], kr[b], gsem[b])
    pltpu.sync_copy(z1d_hbm, st1d_v)
    for ch in range(RT // G):
        pltpu.sync_copy(z2d_hbm, ks_sp.at[pl.ds(t * RT + ch * G, G)])
        pltpu.sync_copy(st1d_v, cnt_sp.at[pl.ds(t * RT + ch * G, G)])
    plsc.subcore_barrier()

    nsup = GPW // NB1

    def body(i, carry):
        for b in range(NB1):
            g = i * NB1 + b
            pltpu.make_async_copy(k_hbm.at[nidx_all.at[g]], kr[b],
                                  gsem[b]).wait()
            pltpu.async_copy(kr[b], ks_sp.at[hidx_all.at[g]], ssem[b],
                             add=True)
            pltpu.async_copy(ones_v, cnt_sp.at[hidx_all.at[g]], ssem[b],
                             add=True)
        for b in range(NB1):
            g = i * NB1 + b
            pltpu.make_async_copy(kr[b], ks_sp.at[hidx_all.at[g]],
                                  ssem[b]).wait()
            pltpu.make_async_copy(ones_v, cnt_sp.at[hidx_all.at[g]],
                                  ssem[b]).wait()

            @pl.when(i < nsup - 1)
            def _(b=b, g=g):
                pltpu.async_copy(k_hbm.at[nidx_all.at[g + NB1]], kr[b],
                                 gsem[b])
        return carry

    lax.fori_loop(0, nsup, body, 0)
    plsc.subcore_barrier()
    for ch in range(RT // G):
        sl = pl.ds(t * RT + ch * G, G)
        pltpu.sync_copy(ks_sp.at[sl], kr[0])
        pltpu.sync_copy(kr[0], ks_out.at[c, sl])
        pltpu.sync_copy(cnt_sp.at[sl], st1d_v)
        pltpu.sync_copy(st1d_v, cnt_out.at[c, sl])


_sc_pass1 = functools.partial(
    pl.kernel,
    out_type=(
        jax.ShapeDtypeStruct((NC, H_PAD, P), jnp.float32),
        jax.ShapeDtypeStruct((NC, H_PAD), jnp.float32),
    ),
    mesh=plsc.VectorSubcoreMesh(core_axis_name="c", subcore_axis_name="s"),
    compiler_params=_SC_PARAMS,
    scratch_types=(
        [pltpu.VMEM_SHARED((H_PAD, P), jnp.float32),
         pltpu.VMEM_SHARED((H_PAD,), jnp.float32),
         pltpu.VMEM((GPW, G), jnp.int32),
         pltpu.VMEM((GPW, G), jnp.int32),
         pltpu.VMEM((G,), jnp.float32),
         pltpu.VMEM((G,), jnp.float32)]
        + [pltpu.VMEM((G, P), jnp.float32)] * NB1
        + [pltpu.SemaphoreType.DMA] * (2 * NB1)
    ),
)(_sc_pass1_body)


def _cent_body(ks_ref, cnt_ref, knm_ref, c_ref, g_ref):
    i = pl.program_id(0)
    ks = ks_ref[0] + ks_ref[1]
    cnt = cnt_ref[0] + cnt_ref[1]
    c = ks / jnp.maximum(cnt, 1.0)[:, None]
    c_ref[...] = c
    mb = jnp.full((1, 128), jnp.max(jnp.sum(c * c, axis=1)))

    @pl.when(i == 0)
    def _():
        g_ref[...] = mb

    @pl.when(jnp.logical_and(i > 0, i < NGB - 1))
    def _():
        g_ref[...] = jnp.maximum(g_ref[...], mb)

    @pl.when(i == NGB - 1)
    def _():
        maxc2 = jnp.maximum(g_ref[...], mb)
        maxk2 = jnp.max(knm_ref[...])
        g_ref[...] = SCALE * (jnp.sqrt(maxc2) + jnp.sqrt(maxk2))


_cent = pl.pallas_call(
    _cent_body,
    grid=(NGB,),
    in_specs=[
        pl.BlockSpec((NC, HB, P), lambda i: (0, i, 0)),
        pl.BlockSpec((NC, HB), lambda i: (0, i)),
        pl.BlockSpec((1, 128), lambda i: (0, 0)),
    ],
    out_specs=[
        pl.BlockSpec((HB, P), lambda i: (i, 0)),
        pl.BlockSpec((1, 128), lambda i: (0, 0)),
    ],
    out_shape=[
        jax.ShapeDtypeStruct((H_PAD, P), jnp.float32),
        jax.ShapeDtypeStruct((1, 128), jnp.float32),
    ],
)


def _sqrt16(x):
    i = plsc.bitcast(x, jnp.int32)
    i = jnp.int32(0x5F3759DF) - (i >> 1)
    y = plsc.bitcast(i, jnp.float32)
    for _ in range(3):
        y = y * (1.5 - 0.5 * x * y * y)
    return jnp.where(x > 0, x * y, 0.0)


def _sc_pass2_body(kv_hbm, c_hbm, g_hbm, nidx_hbm, hidx_hbm, z2d_hbm, z1d_hbm,
                   s_out, den_out,
                   s_sp, den_sp, nidx_all, hidx_all, g_v, st1d_v,
                   kv0, kv1, kv2, kv3, cr0, cr1, cr2, cr3,
                   wv0, wv1, wv2, wv3, ex0, ex1, ex2, ex3,
                   gk0, gk1, gk2, gk3, gc0, gc1, gc2, gc3,
                   ss0, ss1, ss2, ss3):
    kvr = [kv0, kv1, kv2, kv3]
    crr = [cr0, cr1, cr2, cr3]
    wvr = [wv0, wv1, wv2, wv3]
    exr = [ex0, ex1, ex2, ex3]
    gksem = [gk0, gk1, gk2, gk3]
    gcsem = [gc0, gc1, gc2, gc3]
    ssem = [ss0, ss1, ss2, ss3]
    c = lax.axis_index("c")
    t = lax.axis_index("s")
    w = t * NC + c
    pltpu.sync_copy(nidx_hbm.at[pl.ds(w * GPW, GPW)], nidx_all)
    pltpu.sync_copy(hidx_hbm.at[pl.ds(w * GPW, GPW)], hidx_all)
    pltpu.sync_copy(g_hbm, g_v)
    for b in range(NB2):
        pltpu.async_copy(kv_hbm.at[nidx_all.at[b]], kvr[b], gksem[b])
        pltpu.async_copy(c_hbm.at[hidx_all.at[b]], crr[b], gcsem[b])
    pltpu.sync_copy(z1d_hbm, st1d_v)
    for ch in range(RT // G):
        pltpu.sync_copy(z2d_hbm, s_sp.at[pl.ds(t * RT + ch * G, G)])
        pltpu.sync_copy(st1d_v, den_sp.at[pl.ds(t * RT + ch * G, G)])
    plsc.subcore_barrier()

    gvec = g_v[...]
    rows0 = jnp.arange(16, dtype=jnp.int32)
    nsup = GPW // NB2

    def body(i, carry):
        for b in range(NB2):
            g = i * NB2 + b
            pltpu.make_async_copy(kv_hbm.at[nidx_all.at[g]], kvr[b],
                                  gksem[b]).wait()
            pltpu.make_async_copy(c_hbm.at[hidx_all.at[g]], crr[b],
                                  gcsem[b]).wait()

            @pl.when(i > 0)
            def _(b=b, g=g):
                pltpu.make_async_copy(wvr[b], s_sp.at[hidx_all.at[g - NB2]],
                                      ssem[b]).wait()
                pltpu.make_async_copy(exr[b], den_sp.at[hidx_all.at[g - NB2]],
                                      ssem[b]).wait()

            def s8_body(s8, cc2, _kv=kvr[b], _cr=crr[b], _wv=wvr[b],
                        _ex=exr[b]):
                rows = rows0 + s8 * 16
                accs = [jnp.zeros((16,), jnp.float32) for _ in range(8)]
                for j in range(P):
                    colj = (rows0 + j) & (P - 1)
                    kvx = plsc.load_gather(_kv, [rows, colj])
                    cvx = plsc.load_gather(_cr, [rows, colj])
                    dlt = kvx - cvx
                    accs[j % 8] = accs[j % 8] + dlt * dlt
                acc = ((accs[0] + accs[1]) + (accs[2] + accs[3])) + (
                    (accs[4] + accs[5]) + (accs[6] + accs[7]))
                ex = jnp.exp(_sqrt16(acc) * SCALE - gvec)
                _ex[pl.ds(s8 * 16, 16)] = ex
                for j in range(P):
                    colj = (rows0 + j) & (P - 1)
                    vv = plsc.load_gather(_kv, [rows, colj + P])
                    plsc.store_scatter(_wv, [rows, colj], vv * ex)
                return cc2

            lax.fori_loop(0, G // 16, s8_body, 0)

            @pl.when(i < nsup - 1)
            def _(b=b, g=g):
                pltpu.async_copy(kv_hbm.at[nidx_all.at[g + NB2]], kvr[b],
                                 gksem[b])
                pltpu.async_copy(c_hbm.at[hidx_all.at[g + NB2]], crr[b],
                                 gcsem[b])

            pltpu.async_copy(wvr[b], s_sp.at[hidx_all.at[g]], ssem[b],
                             add=True)
            pltpu.async_copy(exr[b], den_sp.at[hidx_all.at[g]], ssem[b],
                             add=True)
        return carry

    lax.fori_loop(0, nsup, body, 0)
    for b in range(NB2):
        g = GPW - NB2 + b
        pltpu.make_async_copy(wvr[b], s_sp.at[hidx_all.at[g]],
                              ssem[b]).wait()
        pltpu.make_async_copy(exr[b], den_sp.at[hidx_all.at[g]],
                              ssem[b]).wait()
    plsc.subcore_barrier()
    for ch in range(RT // G):
        sl = pl.ds(t * RT + ch * G, G)
        pltpu.sync_copy(s_sp.at[sl], wv0)
        pltpu.sync_copy(wv0, s_out.at[c, sl])
        pltpu.sync_copy(den_sp.at[sl], st1d_v)
        pltpu.sync_copy(st1d_v, den_out.at[c, sl])


_sc_pass2 = functools.partial(
    pl.kernel,
    out_type=(
        jax.ShapeDtypeStruct((NC, H_PAD, P), jnp.float32),
        jax.ShapeDtypeStruct((NC, H_PAD), jnp.float32),
    ),
    mesh=plsc.VectorSubcoreMesh(core_axis_name="c", subcore_axis_name="s"),
    compiler_params=_SC_PARAMS,
    scratch_types=(
        [pltpu.VMEM_SHARED((H_PAD, P), jnp.float32),
         pltpu.VMEM_SHARED((H_PAD,), jnp.float32),
         pltpu.VMEM((GPW, G), jnp.int32),
         pltpu.VMEM((GPW, G), jnp.int32),
         pltpu.VMEM((16,), jnp.float32),
         pltpu.VMEM((G,), jnp.float32)]
        + [pltpu.VMEM((G, 2 * P), jnp.float32)] * NB2
        + [pltpu.VMEM((G, P), jnp.float32)] * (2 * NB2)
        + [pltpu.VMEM((G,), jnp.float32)] * NB2
        + [pltpu.SemaphoreType.DMA] * (3 * NB2)
    ),
)(_sc_pass2_body)


def _final_body(s_ref, den_ref, wv_ref, o_ref):
    s = s_ref[0] + s_ref[1]
    den = den_ref[0] + den_ref[1]
    hf = s / jnp.where(den > 0, den, 1.0)[:, None]
    o_ref[...] = jnp.dot(hf, wv_ref[...], preferred_element_type=jnp.float32)


_final = pl.pallas_call(
    _final_body,
    grid=(NGB,),
    in_specs=[
        pl.BlockSpec((NC, HB, P), lambda i: (0, i, 0)),
        pl.BlockSpec((NC, HB), lambda i: (0, i)),
        pl.BlockSpec((P, D), lambda i: (0, 0)),
    ],
    out_specs=pl.BlockSpec((HB, D), lambda i: (i, 0)),
    out_shape=jax.ShapeDtypeStruct((H_PAD, D), jnp.float32),
)


def kernel(node_feats, hyperedge_index, num_hyperedges, W_key, W_value):
    del num_hyperedges
    nidx = hyperedge_index[0].astype(jnp.int32)
    hidx = hyperedge_index[1].astype(jnp.int32)
    pad = E_PAD - E
    spread = jnp.arange(pad, dtype=jnp.int32)
    nidx_p = jnp.concatenate([nidx, spread % N])
    hidx_p = jnp.concatenate([hidx, H + spread % (H_PAD - H)])
    nidx2d = nidx_p.reshape(TG, G)
    hidx2d = hidx_p.reshape(TG, G)
    z2d = jnp.zeros((G, P), jnp.float32)
    z1d = jnp.zeros((G,), jnp.float32)
    ones = jnp.ones((G,), jnp.float32)

    k_arr, kv_arr, knm = _proj(node_feats, W_key, W_value)
    ks2, cnt2 = _sc_pass1(k_arr, nidx2d, hidx2d, z2d, z1d, ones)
    c_arr, gfull = _cent(ks2, cnt2, knm)
    g16 = gfull[0, :16]
    s2, den2 = _sc_pass2(kv_arr, c_arr, g16, nidx2d, hidx2d, z2d, z1d)
    out_pad = _final(s2, den2, W_value)
    return out_pad[:H]
